# baseline (speedup 1.0000x reference)
import functools

import jax
import jax.numpy as jnp
from jax import lax
from jax.experimental import pallas as pl
from jax.experimental.pallas import tpu as pltpu
from jax.experimental.pallas import tpu_sc as plsc

B = 16384
C = 1000
NC = 2
NS = 16
L = 16
NW = NC * NS
RPW = B // NW
GCH = 128
NCH = RPW // GCH
NV = RPW // L

_LN2 = 0.6931471805599453
_SQRT2 = 1.4142135623730951


def _log_f32(x):
    bits = lax.bitcast_convert_type(x, jnp.int32)
    e = (bits >> 23) - 127
    mbits = (bits & 0x007FFFFF) | 0x3F800000
    m = lax.bitcast_convert_type(mbits, jnp.float32)
    big = m > _SQRT2
    m = jnp.where(big, m * 0.5, m)
    e = jnp.where(big, e + 1, e)
    s = (m - 1.0) / (m + 1.0)
    z = s * s
    p = 1.0 / 9.0
    p = 1.0 / 7.0 + z * p
    p = 1.0 / 5.0 + z * p
    p = 1.0 / 3.0 + z * p
    p = 1.0 + z * p
    return e.astype(jnp.float32) * _LN2 + 2.0 * s * p


def _sc_body(a_hbm, b_hbm, g_hbm, cols_hbm, out_hbm,
             cols_v, idx_a, idx_b, idx_g, dat_a, dat_b, dat_g, out_v, sem):
    wid = lax.axis_index("s") * NC + lax.axis_index("c")
    base_row = wid * RPW

    pltpu.sync_copy(cols_hbm, cols_v)

    iot = lax.iota(jnp.int32, L)
    for t, idx_ref in enumerate((idx_a, idx_b, idx_g)):
        col = cols_v[pl.ds(t * L, L)]
        for j in range(NV):
            rows = base_row + (j * L) + iot
            idx_ref[pl.ds(j * L, L)] = rows * C + col

    copies = []
    for arr, idx_ref, dat_ref in ((a_hbm, idx_a, dat_a),
                                  (b_hbm, idx_b, dat_b),
                                  (g_hbm, idx_g, dat_g)):
        for k in range(NCH):
            copies.append(pltpu.async_copy(
                arr.at[idx_ref.at[pl.ds(k * GCH, GCH)]],
                dat_ref.at[pl.ds(k * GCH, GCH)],
                sem))
    for cp in copies:
        cp.wait()

    for j in range(NV):
        sl = pl.ds(j * L, L)
        a = dat_a[sl]
        b = dat_b[sl]
        g = dat_g[sl]
        x = jnp.maximum(1.0 - jnp.exp(g), 1e-8)
        val = a + b - _log_f32(x)
        out_v[sl] = jnp.maximum(val, 0.0)

    pltpu.sync_copy(out_v, out_hbm.at[pl.ds(base_row, RPW)])


@functools.partial(
    pl.kernel,
    out_type=jax.ShapeDtypeStruct((B,), jnp.float32),
    mesh=plsc.VectorSubcoreMesh(core_axis_name="c", subcore_axis_name="s"),
    scratch_types=[
        pltpu.VMEM((3 * L,), jnp.int32),
        pltpu.VMEM((RPW,), jnp.int32),
        pltpu.VMEM((RPW,), jnp.int32),
        pltpu.VMEM((RPW,), jnp.int32),
        pltpu.VMEM((RPW,), jnp.float32),
        pltpu.VMEM((RPW,), jnp.float32),
        pltpu.VMEM((RPW,), jnp.float32),
        pltpu.VMEM((RPW,), jnp.float32),
        pltpu.SemaphoreType.DMA,
    ],
)
def _transition_loss_sc(a_hbm, b_hbm, g_hbm, cols_hbm, out_hbm,
                        cols_v, idx_a, idx_b, idx_g,
                        dat_a, dat_b, dat_g, out_v, sem):
    _sc_body(a_hbm, b_hbm, g_hbm, cols_hbm, out_hbm,
             cols_v, idx_a, idx_b, idx_g, dat_a, dat_b, dat_g, out_v, sem)


def kernel(log_y_alpha, log_y_beta, log_y_gamma,
           alpha_index, beta_index, gamma_index):
    cols = jnp.concatenate([
        jnp.full((L,), alpha_index, dtype=jnp.int32),
        jnp.full((L,), beta_index, dtype=jnp.int32),
        jnp.full((L,), gamma_index, dtype=jnp.int32),
    ])
    return _transition_loss_sc(
        log_y_alpha.reshape(-1),
        log_y_beta.reshape(-1),
        log_y_gamma.reshape(-1),
        cols)


# speedup vs baseline: 1.8515x; 1.8515x over previous
import functools

import jax
import jax.numpy as jnp
from jax import lax
from jax.experimental import pallas as pl
from jax.experimental.pallas import tpu as pltpu

B = 16384
C = 1000
LANES = 128
BLK = 2048
GRID = B // BLK


def _body(cols_ref, a_ref, b_ref, g_ref, out_ref):
    lane_ids = lax.broadcasted_iota(jnp.int32, (BLK, LANES), 1)
    ones = jnp.ones((LANES, 1), dtype=jnp.float32)

    def extract(ref, t):
        lane = lax.rem(cols_ref[t], LANES)
        masked = jnp.where(lane_ids == lane, ref[...], 0.0)
        return jax.lax.dot_general(
            masked, ones, (((1,), (0,)), ((), ())),
            preferred_element_type=jnp.float32)

    a = extract(a_ref, 0)
    b = extract(b_ref, 1)
    g = extract(g_ref, 2)
    x = jnp.maximum(1.0 - jnp.exp(g), 1e-8)
    val = a + b - jnp.log(x)
    out_ref[...] = jnp.maximum(val, 0.0).reshape(BLK)


def _col_block_spec(t):
    return pl.BlockSpec(
        (BLK, LANES), lambda i, cols: (i, lax.div(cols[t], LANES)))


@jax.jit
def _transition_loss_tc(a, b, g, cols):
    return pl.pallas_call(
        _body,
        grid_spec=pltpu.PrefetchScalarGridSpec(
            num_scalar_prefetch=1,
            grid=(GRID,),
            in_specs=[_col_block_spec(0), _col_block_spec(1),
                      _col_block_spec(2)],
            out_specs=pl.BlockSpec((BLK,), lambda i, cols: (i,)),
        ),
        out_shape=jax.ShapeDtypeStruct((B,), jnp.float32),
    )(cols, a, b, g)


def kernel(log_y_alpha, log_y_beta, log_y_gamma,
           alpha_index, beta_index, gamma_index):
    cols = jnp.stack([
        jnp.asarray(alpha_index, dtype=jnp.int32),
        jnp.asarray(beta_index, dtype=jnp.int32),
        jnp.asarray(gamma_index, dtype=jnp.int32),
    ])
    return _transition_loss_tc(log_y_alpha, log_y_beta, log_y_gamma, cols)
